# Initial kernel scaffold; baseline (speedup 1.0000x reference)
import jax
import jax.numpy as jnp
from jax.experimental import pallas as pl


def kernel(x, adj_input):
    raise NotImplementedError("write your pallas kernel here")


# speedup vs baseline: 5.7864x; 5.7864x over previous
import functools

import numpy as np
import jax
import jax.numpy as jnp
from jax import lax
from jax.experimental import pallas as pl
from jax.experimental.pallas import tpu as pltpu
from jax.experimental.pallas import tpu_sc as plsc

_SAMPLE_SIZE = 25
_NCOLS = _SAMPLE_SIZE + 1
_NODES = 50000
_BATCH = 4096
_ADJ_COLS = 65
_FEAT = 256
_NW = 32
_ROWS_PER_W = _BATCH // _NW
_IDS_PER_W = _ROWS_PER_W * _NCOLS
_CHUNK = 128
_NCHUNK = _IDS_PER_W // _CHUNK

_PERM25 = (35, 45, 31, 63, 7, 4, 29, 44, 16, 58, 37, 19, 61, 2, 34,
           5, 30, 42, 3, 39, 56, 22, 6, 54, 18)
_COLS = np.array([0] + [p + 1 for p in _PERM25], dtype=np.int32)

_cache = {}


def _flat_tab():
    if "tab" not in _cache:
        n = np.arange(_BATCH * _NCOLS)
        _cache["tab"] = ((n // _NCOLS) * _ADJ_COLS
                         + _COLS[n % _NCOLS]).astype(np.int32)
    return _cache["tab"]


def _build_gather():
    if "call" in _cache:
        return _cache["call"]

    mesh = plsc.VectorSubcoreMesh(core_axis_name="c", subcore_axis_name="s")

    @functools.partial(
        pl.kernel,
        mesh=mesh,
        out_type=jax.ShapeDtypeStruct((_BATCH * _NCOLS, _FEAT), jnp.float32),
        scratch_types=[
            pltpu.VMEM((_IDS_PER_W,), jnp.int32),
            pltpu.VMEM((_IDS_PER_W,), jnp.int32),
            pltpu.VMEM((_CHUNK, _FEAT), jnp.float32),
            pltpu.VMEM((_CHUNK, _FEAT), jnp.float32),
            pltpu.SemaphoreType.DMA,
            pltpu.SemaphoreType.DMA,
            pltpu.SemaphoreType.DMA,
            pltpu.SemaphoreType.DMA,
            pltpu.SemaphoreType.DMA,
        ],
    )
    def _impl(x_hbm, adj_hbm, tab_hbm, out_hbm,
              tab_v, ids_v, buf0, buf1, asem, g0, g1, w0, w1):
        wid = lax.axis_index("s") * 2 + lax.axis_index("c")
        out_base = wid * _IDS_PER_W

        pltpu.sync_copy(tab_hbm.at[pl.ds(out_base, _IDS_PER_W)], tab_v)

        for c in range(_NCHUNK):
            sl = pl.ds(c * _CHUNK, _CHUNK)
            pltpu.make_async_copy(
                adj_hbm.at[tab_v.at[sl]], ids_v.at[sl], asem).start()
        pltpu.make_async_copy(adj_hbm.at[tab_v], ids_v, asem).wait()

        buf = (buf0, buf1)
        gsem = (g0, g1)
        wsem = (w0, w1)

        def g_copy(c, b):
            return pltpu.make_async_copy(
                x_hbm.at[ids_v.at[pl.ds(c * _CHUNK, _CHUNK)]],
                buf[b], gsem[b])

        def w_copy(c, b):
            return pltpu.make_async_copy(
                buf[b],
                out_hbm.at[pl.ds(out_base + c * _CHUNK, _CHUNK)], wsem[b])

        g_copy(0, 0).start()
        g_copy(1, 1).start()

        for c in range(_NCHUNK):
            b = c & 1
            g_copy(c, b).wait()
            w_copy(c, b).start()
            if 1 <= c and c + 1 < _NCHUNK:
                pb = (c - 1) & 1
                w_copy(c - 1, pb).wait()
                g_copy(c + 1, pb).start()

        w_copy(_NCHUNK - 2, 0).wait()
        w_copy(_NCHUNK - 1, 1).wait()

    _cache["call"] = _impl
    return _impl


def kernel(x, adj_input):
    x2 = x.reshape(_NODES, _FEAT)
    adj = adj_input.astype(jnp.int32).reshape(_BATCH * _ADJ_COLS)
    tab = jnp.asarray(_flat_tab())
    out = _build_gather()(x2, adj, tab)
    return out.reshape(_BATCH, _NCOLS * 4, 8, 8)


# speedup vs baseline: 18.2413x; 3.1524x over previous
import functools

import numpy as np
import jax
import jax.numpy as jnp
from jax import lax
from jax.experimental import pallas as pl
from jax.experimental.pallas import tpu as pltpu
from jax.experimental.pallas import tpu_sc as plsc

_SAMPLE_SIZE = 25
_NCOLS = _SAMPLE_SIZE + 1
_NODES = 50000
_BATCH = 4096
_ADJ_COLS = 65
_FEAT = 256
_NW = 32
_ROWS_PER_W = _BATCH // _NW
_IDS_PER_W = _ROWS_PER_W * _NCOLS
_CHUNK = 128
_NCHUNK = _IDS_PER_W // _CHUNK

_PERM25 = (35, 45, 31, 63, 7, 4, 29, 44, 16, 58, 37, 19, 61, 2, 34,
           5, 30, 42, 3, 39, 56, 22, 6, 54, 18)
_COLS = np.array([0] + [p + 1 for p in _PERM25], dtype=np.int32)

_cache = {}


def _flat_tab():
    if "tab" not in _cache:
        w, k, bi = np.meshgrid(np.arange(_NW), np.arange(_NCOLS),
                               np.arange(_ROWS_PER_W), indexing="ij")
        _cache["tab"] = ((w * _ROWS_PER_W + bi) * _ADJ_COLS
                         + _COLS[k]).reshape(-1).astype(np.int32)
    return _cache["tab"]


def _build_gather():
    if "call" in _cache:
        return _cache["call"]

    mesh = plsc.VectorSubcoreMesh(core_axis_name="c", subcore_axis_name="s")

    @functools.partial(
        pl.kernel,
        mesh=mesh,
        out_type=jax.ShapeDtypeStruct((_NCOLS, _BATCH, _FEAT), jnp.float32),
        scratch_types=[
            pltpu.VMEM((_IDS_PER_W,), jnp.int32),
            pltpu.VMEM((_IDS_PER_W,), jnp.int32),
            pltpu.VMEM((_CHUNK, _FEAT), jnp.float32),
            pltpu.VMEM((_CHUNK, _FEAT), jnp.float32),
            pltpu.SemaphoreType.DMA,
            pltpu.SemaphoreType.DMA,
            pltpu.SemaphoreType.DMA,
            pltpu.SemaphoreType.DMA,
            pltpu.SemaphoreType.DMA,
        ],
    )
    def _impl(x_hbm, adj_hbm, tab_hbm, out_hbm,
              tab_v, ids_v, buf0, buf1, asem, g0, g1, w0, w1):
        wid = lax.axis_index("s") * 2 + lax.axis_index("c")
        b0 = wid * _ROWS_PER_W

        pltpu.sync_copy(tab_hbm.at[pl.ds(wid * _IDS_PER_W, _IDS_PER_W)],
                        tab_v)

        for c in range(_NCHUNK):
            sl = pl.ds(c * _CHUNK, _CHUNK)
            pltpu.make_async_copy(
                adj_hbm.at[tab_v.at[sl]], ids_v.at[sl], asem).start()
        pltpu.make_async_copy(adj_hbm.at[tab_v], ids_v, asem).wait()

        buf = (buf0, buf1)
        gsem = (g0, g1)
        wsem = (w0, w1)

        def g_copy(c, b):
            return pltpu.make_async_copy(
                x_hbm.at[ids_v.at[pl.ds(c * _CHUNK, _CHUNK)]],
                buf[b], gsem[b])

        def w_copy(c, b):
            return pltpu.make_async_copy(
                buf[b],
                out_hbm.at[c, pl.ds(b0, _ROWS_PER_W)], wsem[b])

        g_copy(0, 0).start()
        g_copy(1, 1).start()

        for c in range(_NCHUNK):
            b = c & 1
            g_copy(c, b).wait()
            w_copy(c, b).start()
            if 1 <= c and c + 1 < _NCHUNK:
                pb = (c - 1) & 1
                w_copy(c - 1, pb).wait()
                g_copy(c + 1, pb).start()

        w_copy(_NCHUNK - 2, 0).wait()
        w_copy(_NCHUNK - 1, 1).wait()

    _cache["call"] = _impl
    return _impl


def _fmt_body(g_ref, o_ref):
    blk = g_ref[...].reshape(_ROWS_PER_W, _FEAT)
    o_ref[...] = jnp.swapaxes(blk, 0, 1).reshape(4, 8, 1, 8, _ROWS_PER_W)


def _build_format():
    if "fmt" not in _cache:
        _cache["fmt"] = pl.pallas_call(
            _fmt_body,
            grid=(_NCOLS, _BATCH // _ROWS_PER_W),
            in_specs=[pl.BlockSpec((1, _ROWS_PER_W, _FEAT),
                                   lambda k, t: (k, t, 0))],
            out_specs=pl.BlockSpec((4, 8, 1, 8, _ROWS_PER_W),
                                   lambda k, t: (k, 0, t, 0, 0)),
            out_shape=jax.ShapeDtypeStruct(
                (_NCOLS * 4, 8, _BATCH // _ROWS_PER_W, 8, _ROWS_PER_W),
                jnp.float32),
        )
    return _cache["fmt"]


def kernel(x, adj_input):
    x2 = x.reshape(_NODES, _FEAT)
    adj = adj_input.astype(jnp.int32).reshape(_BATCH * _ADJ_COLS)
    tab = jnp.asarray(_flat_tab())
    g = _build_gather()(x2, adj, tab)
    o5 = _build_format()(g)
    return jnp.transpose(o5, (2, 4, 0, 1, 3)).reshape(_BATCH, _NCOLS * 4, 8, 8)


# speedup vs baseline: 47.5913x; 2.6090x over previous
import functools

import numpy as np
import jax
import jax.numpy as jnp
from jax import lax
from jax.experimental import pallas as pl
from jax.experimental.pallas import tpu as pltpu
from jax.experimental.pallas import tpu_sc as plsc

_SAMPLE_SIZE = 25
_NCOLS = _SAMPLE_SIZE + 1
_NODES = 50000
_BATCH = 4096
_ADJ_COLS = 65
_FEAT = 256
_NW = 32
_ROWS_PER_W = _BATCH // _NW
_IDS_PER_W = _ROWS_PER_W * _NCOLS
_CHUNK = 128
_NCHUNK = _IDS_PER_W // _CHUNK

_PERM25 = (35, 45, 31, 63, 7, 4, 29, 44, 16, 58, 37, 19, 61, 2, 34,
           5, 30, 42, 3, 39, 56, 22, 6, 54, 18)
_COLS = np.array([0] + [p + 1 for p in _PERM25], dtype=np.int32)

_cache = {}


def _flat_tab():
    if "tab" not in _cache:
        w, k, bi = np.meshgrid(np.arange(_NW), np.arange(_NCOLS),
                               np.arange(_ROWS_PER_W), indexing="ij")
        _cache["tab"] = ((w * _ROWS_PER_W + bi) * _ADJ_COLS
                         + _COLS[k]).reshape(-1).astype(np.int32)
    return _cache["tab"]


def _build_gather():
    if "call" in _cache:
        return _cache["call"]

    mesh = plsc.VectorSubcoreMesh(core_axis_name="c", subcore_axis_name="s")

    @functools.partial(
        pl.kernel,
        mesh=mesh,
        out_type=jax.ShapeDtypeStruct((_NCOLS, _BATCH, _FEAT), jnp.float32),
        scratch_types=[
            pltpu.VMEM((_IDS_PER_W,), jnp.int32),
            pltpu.VMEM((_IDS_PER_W,), jnp.int32),
            pltpu.VMEM((_CHUNK, _FEAT), jnp.float32),
            pltpu.VMEM((_CHUNK, _FEAT), jnp.float32),
            pltpu.SemaphoreType.DMA,
            pltpu.SemaphoreType.DMA,
            pltpu.SemaphoreType.DMA,
            pltpu.SemaphoreType.DMA,
            pltpu.SemaphoreType.DMA,
        ],
    )
    def _impl(x_hbm, adj_hbm, tab_hbm, out_hbm,
              tab_v, ids_v, buf0, buf1, asem, g0, g1, w0, w1):
        wid = lax.axis_index("s") * 2 + lax.axis_index("c")
        b0 = wid * _ROWS_PER_W

        pltpu.sync_copy(tab_hbm.at[pl.ds(wid * _IDS_PER_W, _IDS_PER_W)],
                        tab_v)

        for c in range(_NCHUNK):
            sl = pl.ds(c * _CHUNK, _CHUNK)
            pltpu.make_async_copy(
                adj_hbm.at[tab_v.at[sl]], ids_v.at[sl], asem).start()
        pltpu.make_async_copy(adj_hbm.at[tab_v], ids_v, asem).wait()

        buf = (buf0, buf1)
        gsem = (g0, g1)
        wsem = (w0, w1)

        def g_copy(c, b):
            return pltpu.make_async_copy(
                x_hbm.at[ids_v.at[pl.ds(c * _CHUNK, _CHUNK)]],
                buf[b], gsem[b])

        def w_copy(c, b):
            return pltpu.make_async_copy(
                buf[b],
                out_hbm.at[c, pl.ds(b0, _ROWS_PER_W)], wsem[b])

        g_copy(0, 0).start()
        g_copy(1, 1).start()

        for c in range(_NCHUNK):
            b = c & 1
            g_copy(c, b).wait()
            w_copy(c, b).start()
            if 1 <= c and c + 1 < _NCHUNK:
                pb = (c - 1) & 1
                w_copy(c - 1, pb).wait()
                g_copy(c + 1, pb).start()

        w_copy(_NCHUNK - 2, 0).wait()
        w_copy(_NCHUNK - 1, 1).wait()

    _cache["call"] = _impl
    return _impl


_KBLK = 13


def _fmt_body(g_ref, o_ref):
    for kk in range(_KBLK):
        blk = g_ref[kk]
        o_ref[pl.ds(kk * 4, 4)] = jnp.swapaxes(blk, 0, 1).reshape(
            4, 8, 1, 8, _ROWS_PER_W)


def _build_format():
    if "fmt" not in _cache:
        _cache["fmt"] = pl.pallas_call(
            _fmt_body,
            grid=(_NCOLS // _KBLK, _BATCH // _ROWS_PER_W),
            in_specs=[pl.BlockSpec((_KBLK, _ROWS_PER_W, _FEAT),
                                   lambda k, t: (k, t, 0))],
            out_specs=pl.BlockSpec((4 * _KBLK, 8, 1, 8, _ROWS_PER_W),
                                   lambda k, t: (k, 0, t, 0, 0)),
            out_shape=jax.ShapeDtypeStruct(
                (_NCOLS * 4, 8, _BATCH // _ROWS_PER_W, 8, _ROWS_PER_W),
                jnp.float32),
        )
    return _cache["fmt"]


def kernel(x, adj_input):
    x2 = x.reshape(_NODES, _FEAT)
    adj = adj_input.astype(jnp.int32).reshape(_BATCH * _ADJ_COLS)
    tab = jnp.asarray(_flat_tab())
    g = _build_gather()(x2, adj, tab)
    o5 = _build_format()(g)
    return jnp.transpose(o5, (2, 4, 0, 1, 3)).reshape(_BATCH, _NCOLS * 4, 8, 8)
